# baseline (speedup 1.0000x reference)
import functools

import jax
import jax.numpy as jnp
from jax import lax
from jax.experimental import pallas as pl
from jax.experimental.pallas import tpu as pltpu
from jax.experimental.pallas import tpu_sc as plsc

NUM_CORES = 2
NUM_TILES = 16


def _pick_chunk(ept):
    for k in (128, 120, 112, 104, 96, 88, 80, 72, 64, 56, 48, 40, 32, 24, 16, 8):
        if ept % k == 0:
            return k
    raise ValueError(f"edges-per-tile {ept} not divisible by a multiple of 8")


def _sc_mesh():
    return plsc.VectorSubcoreMesh(
        core_axis_name="c", subcore_axis_name="s",
        num_cores=NUM_CORES, num_subcores=NUM_TILES)


def _make_degree(N, E):
    ept = E // NUM_TILES
    K = _pick_chunk(ept)
    nch = ept // K
    slab = (N // NUM_TILES) & ~7
    rem = N - slab * NUM_TILES

    assert nch % 2 == 0

    @functools.partial(
        pl.kernel,
        out_type=jax.ShapeDtypeStruct((NUM_CORES * N, 16), jnp.float32),
        mesh=_sc_mesh(),
        scratch_types=[
            pltpu.VMEM_SHARED((N, 16), jnp.float32),
            pltpu.VMEM((nch, K), jnp.int32),
            pltpu.VMEM((K, 16), jnp.float32),
            pltpu.SemaphoreType.DMA,
        ],
    )
    def deg_kernel(dst_hbm, ones_hbm, zeros_hbm, out_hbm, acc, dst_all,
                   ones_v, sem_a):
        c = lax.axis_index("c")
        s = lax.axis_index("s")
        wid = c * NUM_TILES + s
        pltpu.sync_copy(zeros_hbm.at[pl.ds(0, slab)],
                        acc.at[pl.ds(s * slab, slab)])
        if rem:
            @pl.when(s == 0)
            def _zrem():
                pltpu.sync_copy(zeros_hbm.at[pl.ds(0, rem)],
                                acc.at[pl.ds(NUM_TILES * slab, rem)])
        pltpu.sync_copy(ones_hbm, ones_v)
        pltpu.sync_copy(dst_hbm.at[wid], dst_all)
        plsc.subcore_barrier()

        W = next(w for w in (10, 8, 5, 4, 2, 1) if nch % w == 0)

        def _drain():
            for w in range(W):
                pltpu.make_async_copy(ones_v, acc.at[dst_all.at[0]],
                                      sem_a).wait()

        for w in range(W):
            pltpu.async_copy(ones_v, acc.at[dst_all.at[w]], sem_a, add=True)

        @pl.loop(W, nch, step=W)
        def _window(j):
            for w in range(W):
                pltpu.async_copy(ones_v, acc.at[dst_all.at[j + w]], sem_a,
                                 add=True)
            _drain()

        _drain()

        plsc.subcore_barrier()
        pltpu.sync_copy(acc.at[pl.ds(s * slab, slab)],
                        out_hbm.at[pl.ds(c * N + s * slab, slab)])
        if rem:
            @pl.when(s == 0)
            def _orem():
                pltpu.sync_copy(
                    acc.at[pl.ds(NUM_TILES * slab, rem)],
                    out_hbm.at[pl.ds(c * N + NUM_TILES * slab, rem)])

    return deg_kernel


def _make_segsum(N, E, D):
    ept = E // NUM_TILES
    K = _pick_chunk(ept)
    nch = ept // K
    slab = (N // NUM_TILES) & ~7
    rem = N - slab * NUM_TILES
    assert nch % 3 == 1 and nch >= 7, nch

    @functools.partial(
        pl.kernel,
        out_type=jax.ShapeDtypeStruct((NUM_CORES * N, D), jnp.float32),
        mesh=_sc_mesh(),
        scratch_types=(
            [pltpu.VMEM_SHARED((N, D), jnp.float32)]
            + [pltpu.VMEM((2, K), jnp.int32)] * 3
            + [pltpu.VMEM((K, D), jnp.float32)] * 3
            + [pltpu.SemaphoreType.DMA] * 9
        ),
    )
    def seg_kernel(ys_hbm, comb_hbm, zeros_hbm, out_hbm, acc,
                   idx0, idx1, idx2, buf0, buf1, buf2,
                   si0, si1, si2, sg0, sg1, sg2, ss0, ss1, ss2):
        c = lax.axis_index("c")
        s = lax.axis_index("s")
        pltpu.sync_copy(zeros_hbm.at[pl.ds(0, slab)],
                        acc.at[pl.ds(s * slab, slab)])
        if rem:
            @pl.when(s == 0)
            def _zrem():
                pltpu.sync_copy(zeros_hbm.at[pl.ds(0, rem)],
                                acc.at[pl.ds(NUM_TILES * slab, rem)])
        wid = c * NUM_TILES + s

        idxs = (idx0, idx1, idx2)
        bufs = (buf0, buf1, buf2)
        sis = (si0, si1, si2)
        sgs = (sg0, sg1, sg2)
        sss = (ss0, ss1, ss2)

        def idx_descs(j, l):
            return (pltpu.make_async_copy(comb_hbm.at[wid, j], idxs[l],
                                          sis[l]),)

        def g_desc(l):
            return pltpu.make_async_copy(ys_hbm.at[idxs[l].at[0]], bufs[l],
                                         sgs[l])

        def s_fire(l):
            pltpu.async_copy(bufs[l], acc.at[idxs[l].at[1]], sss[l],
                             add=True)

        def s_wait(l):
            pltpu.make_async_copy(bufs[l], acc.at[idxs[l].at[1]],
                                  sss[l]).wait()

        def chunk(j, l, first=False, prefetch=True):
            m = (l + 2) % 3
            if not first:
                s_wait(m)
            if prefetch:
                for d in idx_descs(j + 2, m):
                    d.start()
            g_desc(l).wait()
            s_fire(l)
            if prefetch:
                for d in idx_descs(j + 2, m):
                    d.wait()
                g_desc(m).start()

        for d in idx_descs(0, 0):
            d.start()
        for d in idx_descs(1, 1):
            d.start()
        plsc.subcore_barrier()
        for d in idx_descs(0, 0):
            d.wait()
        g_desc(0).start()
        for d in idx_descs(1, 1):
            d.wait()
        g_desc(1).start()

        chunk(0, 0, first=True)

        @pl.loop(0, (nch - 7) // 3)
        def _trip(t):
            j = 1 + 3 * t
            chunk(j, 1)
            chunk(j + 1, 2)
            chunk(j + 2, 0)

        for j in range(nch - 6, nch):
            chunk(j, j % 3, prefetch=(j + 2 < nch))
        s_wait((nch - 1) % 3)

        plsc.subcore_barrier()
        pltpu.sync_copy(acc.at[pl.ds(s * slab, slab)],
                        out_hbm.at[pl.ds(c * N + s * slab, slab)])
        if rem:
            @pl.when(s == 0)
            def _orem():
                pltpu.sync_copy(
                    acc.at[pl.ds(NUM_TILES * slab, rem)],
                    out_hbm.at[pl.ds(c * N + NUM_TILES * slab, rem)])

    return seg_kernel


def _mm_prescale_body(x_ref, w_ref, deg_ref, o_ref):
    dinv = lax.rsqrt(deg_ref[:, 0:1] + 1.0)
    o_ref[...] = jnp.dot(x_ref[...], w_ref[...],
                         preferred_element_type=jnp.float32) * dinv


def _mm_prescale(x, w, deg16, br):
    n2, d = x.shape
    grid = (n2 // br,)
    return pl.pallas_call(
        _mm_prescale_body,
        grid=grid,
        in_specs=[
            pl.BlockSpec((br, d), lambda i: (i, 0)),
            pl.BlockSpec((d, w.shape[1]), lambda i: (0, 0)),
            pl.BlockSpec((br, 16), lambda i: (i, 0)),
        ],
        out_specs=pl.BlockSpec((br, w.shape[1]), lambda i: (i, 0)),
        out_shape=jax.ShapeDtypeStruct((n2, w.shape[1]), jnp.float32),
    )(x, w, deg16)


def _post_mm_body(s_ref, ys_ref, b_ref, w_ref, deg_ref, o_ref):
    dinv = lax.rsqrt(deg_ref[:, 0:1] + 1.0)
    h = jnp.maximum(dinv * (s_ref[...] + ys_ref[...]) + b_ref[...], 0.0)
    o_ref[...] = jnp.dot(h, w_ref[...],
                         preferred_element_type=jnp.float32) * dinv


def _post_mm(s1, ys1, b1, w2, deg16, br):
    n2, d = s1.shape
    grid = (n2 // br,)
    return pl.pallas_call(
        _post_mm_body,
        grid=grid,
        in_specs=[
            pl.BlockSpec((br, d), lambda i: (i, 0)),
            pl.BlockSpec((br, d), lambda i: (i, 0)),
            pl.BlockSpec((1, d), lambda i: (0, 0)),
            pl.BlockSpec((d, w2.shape[1]), lambda i: (0, 0)),
            pl.BlockSpec((br, 16), lambda i: (i, 0)),
        ],
        out_specs=pl.BlockSpec((br, w2.shape[1]), lambda i: (i, 0)),
        out_shape=jax.ShapeDtypeStruct((n2, w2.shape[1]), jnp.float32),
    )(s1, ys1, b1, w2, deg16)


def _l2n(x):
    nrm = jnp.sqrt(jnp.sum(x * x, axis=1, keepdims=True))
    return x / jnp.maximum(nrm, 1e-12)


def _flash_body(N, BR, CB, s2_ref, ys2_ref, b2_ref, deg_ref, t_ref,
                out_ref, vgn_ref, tsc_ref, acc_ref):
    i = pl.program_id(0)
    nb = pl.num_programs(0)
    invt = 1.0 / t_ref[0, 0]
    M = jnp.minimum(jnp.abs(invt), 50.0)

    @pl.when(i == 0)
    def _init():
        dinv_v = lax.rsqrt(deg_ref[N:, 0:1] + 1.0)
        ev = dinv_v * (s2_ref[N:, :] + ys2_ref[N:, :]) + b2_ref[...]
        vgn_ref[...] = _l2n(ev).astype(jnp.bfloat16)
        acc_ref[0] = 0.0

    rows = pl.ds(i * BR, BR)
    dinv_t = lax.rsqrt(deg_ref[rows, 0:1] + 1.0)
    et = dinv_t * (s2_ref[rows, :] + ys2_ref[rows, :]) + b2_ref[...]
    tn = _l2n(et)
    tsc_ref[...] = (tn * invt).astype(jnp.bfloat16)
    vg_diag = vgn_ref[rows, :].astype(jnp.float32)
    picked = jnp.clip(jnp.sum(tn * vg_diag, axis=1, keepdims=True) * invt,
                      -50.0, 50.0)

    ones_col = jnp.ones((CB, 1), jnp.float32)

    @pl.loop(0, N // CB, init_carry=jnp.zeros((BR, 1), jnp.float32))
    def col_loop(c, rowsum):
        vc = vgn_ref[pl.ds(c * CB, CB), :]
        simc = lax.dot_general(tsc_ref[...], vc, (((1,), (1,)), ((), ())),
                               preferred_element_type=jnp.float32)
        e = jnp.exp(jnp.clip(simc, -50.0, 50.0) - M)
        return rowsum + jnp.dot(e, ones_col,
                                preferred_element_type=jnp.float32)

    logz = M + jnp.log(col_loop)
    acc_ref[0] += jnp.sum(logz - picked)

    @pl.when(i == nb - 1)
    def _fin():
        out_ref[0, 0] = acc_ref[0] * (1.0 / N)


def _flash_loss(s2, ys2, b2, deg16, temp, N, BR, CB):
    d = s2.shape[1]
    grid = (N // BR,)
    return pl.pallas_call(
        functools.partial(_flash_body, N, BR, CB),
        grid=grid,
        in_specs=[
            pl.BlockSpec((2 * N, d), lambda i: (0, 0)),
            pl.BlockSpec((2 * N, d), lambda i: (0, 0)),
            pl.BlockSpec((1, d), lambda i: (0, 0)),
            pl.BlockSpec((2 * N, 16), lambda i: (0, 0)),
            pl.BlockSpec(memory_space=pltpu.SMEM),
        ],
        out_specs=pl.BlockSpec((1, 1), lambda i: (0, 0),
                               memory_space=pltpu.SMEM),
        out_shape=jax.ShapeDtypeStruct((1, 1), jnp.float32),
        scratch_shapes=[
            pltpu.VMEM((N, d), jnp.bfloat16),
            pltpu.VMEM((BR, d), jnp.bfloat16),
            pltpu.SMEM((1,), jnp.float32),
        ],
    )(s2, ys2, b2, deg16, temp)


def kernel(tg_x, tg_edge_index, vg_x, vg_edge_index, W1, b1, W2, b2,
           temperature):
    N, D = tg_x.shape
    E = tg_edge_index.shape[1]

    X = jnp.concatenate([tg_x, vg_x], axis=0)
    src_g = jnp.concatenate([tg_edge_index[0], vg_edge_index[0] + N])
    dst_l = jnp.concatenate([tg_edge_index[1], vg_edge_index[1]])

    npt = N // NUM_TILES
    ept = E // NUM_TILES
    Kd = _pick_chunk(ept)
    dst3 = dst_l.reshape(NUM_CORES * NUM_TILES, ept // Kd, Kd)
    zeros16 = jnp.zeros((npt, 16), jnp.float32)
    ones16 = jnp.ones((Kd, 16), jnp.float32)
    zerosD = jnp.zeros((npt, D), jnp.float32)
    b1r = b1.reshape(1, -1)
    b2r = b2.reshape(1, -1)
    tempr = jnp.asarray(temperature, jnp.float32).reshape(1, 1)

    nchs = ept // Kd
    comb = jnp.concatenate(
        [src_g.reshape(NUM_CORES * NUM_TILES, nchs, 1, Kd),
         dst_l.reshape(NUM_CORES * NUM_TILES, nchs, 1, Kd)], axis=2)

    deg16 = _make_degree(N, E)(dst3, ones16, zeros16)
    segsum = _make_segsum(N, E, D)

    ys1 = _mm_prescale(X, W1, deg16, br=1000)
    s1 = segsum(ys1, comb, zerosD)
    ys2 = _post_mm(s1, ys1, b1r, W2, deg16, br=1000)
    s2 = segsum(ys2, comb, zerosD)
    loss = _flash_loss(s2, ys2, b2r, deg16, tempr, N, BR=400, CB=2000)
    return loss[0, 0]


# speedup vs baseline: 1.0004x; 1.0004x over previous
import functools

import jax
import jax.numpy as jnp
from jax import lax
from jax.experimental import pallas as pl
from jax.experimental.pallas import tpu as pltpu
from jax.experimental.pallas import tpu_sc as plsc

NUM_CORES = 2
NUM_TILES = 16


def _pick_chunk(ept):
    for k in (128, 120, 112, 104, 96, 88, 80, 72, 64, 56, 48, 40, 32, 24, 16, 8):
        if ept % k == 0:
            return k
    raise ValueError(f"edges-per-tile {ept} not divisible by a multiple of 8")


def _sc_mesh():
    return plsc.VectorSubcoreMesh(
        core_axis_name="c", subcore_axis_name="s",
        num_cores=NUM_CORES, num_subcores=NUM_TILES)


def _make_degree(N, E):
    ept = E // NUM_TILES
    K = _pick_chunk(ept)
    nch = ept // K
    slab = (N // NUM_TILES) & ~7
    rem = N - slab * NUM_TILES

    assert nch % 2 == 0

    @functools.partial(
        pl.kernel,
        out_type=jax.ShapeDtypeStruct((NUM_CORES * N, 16), jnp.float32),
        mesh=_sc_mesh(),
        scratch_types=[
            pltpu.VMEM_SHARED((N, 16), jnp.float32),
            pltpu.VMEM((nch, K), jnp.int32),
            pltpu.VMEM((K, 16), jnp.float32),
            pltpu.SemaphoreType.DMA,
        ],
    )
    def deg_kernel(dst_hbm, ones_hbm, zeros_hbm, out_hbm, acc, dst_all,
                   ones_v, sem_a):
        c = lax.axis_index("c")
        s = lax.axis_index("s")
        wid = c * NUM_TILES + s
        pltpu.sync_copy(zeros_hbm.at[pl.ds(0, slab)],
                        acc.at[pl.ds(s * slab, slab)])
        if rem:
            @pl.when(s == 0)
            def _zrem():
                pltpu.sync_copy(zeros_hbm.at[pl.ds(0, rem)],
                                acc.at[pl.ds(NUM_TILES * slab, rem)])
        pltpu.sync_copy(ones_hbm, ones_v)
        pltpu.sync_copy(dst_hbm.at[wid], dst_all)
        plsc.subcore_barrier()

        W = next(w for w in (10, 8, 5, 4, 2, 1) if nch % w == 0)

        def _drain():
            for w in range(W):
                pltpu.make_async_copy(ones_v, acc.at[dst_all.at[0]],
                                      sem_a).wait()

        for w in range(W):
            pltpu.async_copy(ones_v, acc.at[dst_all.at[w]], sem_a, add=True)

        @pl.loop(W, nch, step=W)
        def _window(j):
            for w in range(W):
                pltpu.async_copy(ones_v, acc.at[dst_all.at[j + w]], sem_a,
                                 add=True)
            _drain()

        _drain()

        plsc.subcore_barrier()
        pltpu.sync_copy(acc.at[pl.ds(s * slab, slab)],
                        out_hbm.at[pl.ds(c * N + s * slab, slab)])
        if rem:
            @pl.when(s == 0)
            def _orem():
                pltpu.sync_copy(
                    acc.at[pl.ds(NUM_TILES * slab, rem)],
                    out_hbm.at[pl.ds(c * N + NUM_TILES * slab, rem)])

    return deg_kernel


def _make_segsum(N, E, D):
    ept = E // NUM_TILES
    K = _pick_chunk(ept)
    nch = ept // K
    slab = (N // NUM_TILES) & ~7
    rem = N - slab * NUM_TILES
    assert nch % 3 == 1 and nch >= 7, nch

    @functools.partial(
        pl.kernel,
        out_type=jax.ShapeDtypeStruct((NUM_CORES * N, D), jnp.float32),
        mesh=_sc_mesh(),
        scratch_types=(
            [pltpu.VMEM_SHARED((N, D), jnp.float32)]
            + [pltpu.VMEM((2, K), jnp.int32)] * 3
            + [pltpu.VMEM((K, D), jnp.float32)] * 3
            + [pltpu.SemaphoreType.DMA] * 9
        ),
    )
    def seg_kernel(ys_hbm, comb_hbm, zeros_hbm, out_hbm, acc,
                   idx0, idx1, idx2, buf0, buf1, buf2,
                   si0, si1, si2, sg0, sg1, sg2, ss0, ss1, ss2):
        c = lax.axis_index("c")
        s = lax.axis_index("s")
        pltpu.sync_copy(zeros_hbm.at[pl.ds(0, slab)],
                        acc.at[pl.ds(s * slab, slab)])
        if rem:
            @pl.when(s == 0)
            def _zrem():
                pltpu.sync_copy(zeros_hbm.at[pl.ds(0, rem)],
                                acc.at[pl.ds(NUM_TILES * slab, rem)])
        wid = c * NUM_TILES + s

        idxs = (idx0, idx1, idx2)
        bufs = (buf0, buf1, buf2)
        sis = (si0, si1, si2)
        sgs = (sg0, sg1, sg2)
        sss = (ss0, ss1, ss2)

        def idx_descs(j, l):
            return (pltpu.make_async_copy(comb_hbm.at[wid, j], idxs[l],
                                          sis[l]),)

        def g_desc(l):
            return pltpu.make_async_copy(ys_hbm.at[idxs[l].at[0]], bufs[l],
                                         sgs[l])

        def s_fire(l):
            pltpu.async_copy(bufs[l], acc.at[idxs[l].at[1]], sss[l],
                             add=True)

        def s_wait(l):
            pltpu.make_async_copy(bufs[l], acc.at[idxs[l].at[1]],
                                  sss[l]).wait()

        def chunk(j, l, first=False, prefetch=True):
            m = (l + 2) % 3
            if not first:
                s_wait(m)
            if prefetch:
                for d in idx_descs(j + 2, m):
                    d.start()
            g_desc(l).wait()
            s_fire(l)
            if prefetch:
                for d in idx_descs(j + 2, m):
                    d.wait()
                g_desc(m).start()

        for d in idx_descs(0, 0):
            d.start()
        for d in idx_descs(1, 1):
            d.start()
        plsc.subcore_barrier()
        for d in idx_descs(0, 0):
            d.wait()
        g_desc(0).start()
        for d in idx_descs(1, 1):
            d.wait()
        g_desc(1).start()

        chunk(0, 0, first=True)

        @pl.loop(0, (nch - 7) // 3)
        def _trip(t):
            j = 1 + 3 * t
            chunk(j, 1)
            chunk(j + 1, 2)
            chunk(j + 2, 0)

        for j in range(nch - 6, nch):
            chunk(j, j % 3, prefetch=(j + 2 < nch))
        s_wait((nch - 1) % 3)

        plsc.subcore_barrier()
        pltpu.sync_copy(acc.at[pl.ds(s * slab, slab)],
                        out_hbm.at[pl.ds(c * N + s * slab, slab)])
        if rem:
            @pl.when(s == 0)
            def _orem():
                pltpu.sync_copy(
                    acc.at[pl.ds(NUM_TILES * slab, rem)],
                    out_hbm.at[pl.ds(c * N + NUM_TILES * slab, rem)])

    return seg_kernel


def _mm_prescale_body(x_ref, w_ref, deg_ref, o_ref):
    dinv = lax.rsqrt(deg_ref[:, 0:1] + 1.0)
    o_ref[...] = jnp.dot(x_ref[...], w_ref[...],
                         preferred_element_type=jnp.float32) * dinv


def _mm_prescale(x, w, deg16, br):
    n2, d = x.shape
    grid = (n2 // br,)
    return pl.pallas_call(
        _mm_prescale_body,
        grid=grid,
        in_specs=[
            pl.BlockSpec((br, d), lambda i: (i, 0)),
            pl.BlockSpec((d, w.shape[1]), lambda i: (0, 0)),
            pl.BlockSpec((br, 16), lambda i: (i, 0)),
        ],
        out_specs=pl.BlockSpec((br, w.shape[1]), lambda i: (i, 0)),
        out_shape=jax.ShapeDtypeStruct((n2, w.shape[1]), jnp.float32),
    )(x, w, deg16)


def _post_mm_body(s_ref, ys_ref, b_ref, w_ref, deg_ref, o_ref):
    dinv = lax.rsqrt(deg_ref[:, 0:1] + 1.0)
    h = jnp.maximum(dinv * (s_ref[...] + ys_ref[...]) + b_ref[...], 0.0)
    o_ref[...] = jnp.dot(h, w_ref[...],
                         preferred_element_type=jnp.float32) * dinv


def _post_mm(s1, ys1, b1, w2, deg16, br):
    n2, d = s1.shape
    grid = (n2 // br,)
    return pl.pallas_call(
        _post_mm_body,
        grid=grid,
        in_specs=[
            pl.BlockSpec((br, d), lambda i: (i, 0)),
            pl.BlockSpec((br, d), lambda i: (i, 0)),
            pl.BlockSpec((1, d), lambda i: (0, 0)),
            pl.BlockSpec((d, w2.shape[1]), lambda i: (0, 0)),
            pl.BlockSpec((br, 16), lambda i: (i, 0)),
        ],
        out_specs=pl.BlockSpec((br, w2.shape[1]), lambda i: (i, 0)),
        out_shape=jax.ShapeDtypeStruct((n2, w2.shape[1]), jnp.float32),
    )(s1, ys1, b1, w2, deg16)


def _l2n(x):
    nrm = jnp.sqrt(jnp.sum(x * x, axis=1, keepdims=True))
    return x / jnp.maximum(nrm, 1e-12)


def _flash_body(N, BR, CB, s2_ref, ys2_ref, b2_ref, deg_ref, t_ref,
                out_ref, vgn_ref, tsc_ref, acc_ref):
    i = pl.program_id(0)
    nb = pl.num_programs(0)
    invt = 1.0 / t_ref[0, 0]
    M = jnp.minimum(jnp.abs(invt), 50.0)

    @pl.when(i == 0)
    def _init():
        dinv_v = lax.rsqrt(deg_ref[N:, 0:1] + 1.0)
        ev = dinv_v * (s2_ref[N:, :] + ys2_ref[N:, :]) + b2_ref[...]
        vgn_ref[...] = _l2n(ev).astype(jnp.bfloat16)
        acc_ref[0] = 0.0

    rows = pl.ds(i * BR, BR)
    dinv_t = lax.rsqrt(deg_ref[rows, 0:1] + 1.0)
    et = dinv_t * (s2_ref[rows, :] + ys2_ref[rows, :]) + b2_ref[...]
    tn = _l2n(et)
    tsc_ref[...] = (tn * invt).astype(jnp.bfloat16)
    vg_diag = vgn_ref[rows, :].astype(jnp.float32)
    picked = jnp.clip(jnp.sum(tn * vg_diag, axis=1, keepdims=True) * invt,
                      -50.0, 50.0)

    ones_col = jnp.ones((CB, 1), jnp.bfloat16)

    @pl.loop(0, N // CB, init_carry=jnp.zeros((BR, 1), jnp.float32))
    def col_loop(c, rowsum):
        vc = vgn_ref[pl.ds(c * CB, CB), :]
        simc = lax.dot_general(tsc_ref[...], vc, (((1,), (1,)), ((), ())),
                               preferred_element_type=jnp.float32)
        z = (jnp.clip(simc, -50.0, 50.0) - M).astype(jnp.bfloat16)
        e = jnp.exp(z)
        return rowsum + jnp.dot(e, ones_col,
                                preferred_element_type=jnp.float32)

    logz = M + jnp.log(col_loop)
    acc_ref[0] += jnp.sum(logz - picked)

    @pl.when(i == nb - 1)
    def _fin():
        out_ref[0, 0] = acc_ref[0] * (1.0 / N)


def _flash_loss(s2, ys2, b2, deg16, temp, N, BR, CB):
    d = s2.shape[1]
    grid = (N // BR,)
    return pl.pallas_call(
        functools.partial(_flash_body, N, BR, CB),
        grid=grid,
        in_specs=[
            pl.BlockSpec((2 * N, d), lambda i: (0, 0)),
            pl.BlockSpec((2 * N, d), lambda i: (0, 0)),
            pl.BlockSpec((1, d), lambda i: (0, 0)),
            pl.BlockSpec((2 * N, 16), lambda i: (0, 0)),
            pl.BlockSpec(memory_space=pltpu.SMEM),
        ],
        out_specs=pl.BlockSpec((1, 1), lambda i: (0, 0),
                               memory_space=pltpu.SMEM),
        out_shape=jax.ShapeDtypeStruct((1, 1), jnp.float32),
        scratch_shapes=[
            pltpu.VMEM((N, d), jnp.bfloat16),
            pltpu.VMEM((BR, d), jnp.bfloat16),
            pltpu.SMEM((1,), jnp.float32),
        ],
    )(s2, ys2, b2, deg16, temp)


def kernel(tg_x, tg_edge_index, vg_x, vg_edge_index, W1, b1, W2, b2,
           temperature):
    N, D = tg_x.shape
    E = tg_edge_index.shape[1]

    X = jnp.concatenate([tg_x, vg_x], axis=0)
    src_g = jnp.concatenate([tg_edge_index[0], vg_edge_index[0] + N])
    dst_l = jnp.concatenate([tg_edge_index[1], vg_edge_index[1]])

    npt = N // NUM_TILES
    ept = E // NUM_TILES
    Kd = _pick_chunk(ept)
    dst3 = dst_l.reshape(NUM_CORES * NUM_TILES, ept // Kd, Kd)
    zeros16 = jnp.zeros((npt, 16), jnp.float32)
    ones16 = jnp.ones((Kd, 16), jnp.float32)
    zerosD = jnp.zeros((npt, D), jnp.float32)
    b1r = b1.reshape(1, -1)
    b2r = b2.reshape(1, -1)
    tempr = jnp.asarray(temperature, jnp.float32).reshape(1, 1)

    nchs = ept // Kd
    comb = jnp.concatenate(
        [src_g.reshape(NUM_CORES * NUM_TILES, nchs, 1, Kd),
         dst_l.reshape(NUM_CORES * NUM_TILES, nchs, 1, Kd)], axis=2)

    deg16 = _make_degree(N, E)(dst3, ones16, zeros16)
    segsum = _make_segsum(N, E, D)

    ys1 = _mm_prescale(X, W1, deg16, br=1000)
    s1 = segsum(ys1, comb, zerosD)
    ys2 = _post_mm(s1, ys1, b1r, W2, deg16, br=1000)
    s2 = segsum(ys2, comb, zerosD)
    loss = _flash_loss(s2, ys2, b2r, deg16, tempr, N, BR=400, CB=2000)
    return loss[0, 0]


# speedup vs baseline: 1.0283x; 1.0279x over previous
import functools

import jax
import jax.numpy as jnp
from jax import lax
from jax.experimental import pallas as pl
from jax.experimental.pallas import tpu as pltpu
from jax.experimental.pallas import tpu_sc as plsc

NUM_CORES = 2
NUM_TILES = 16


def _pick_chunk(ept):
    for k in (128, 120, 112, 104, 96, 88, 80, 72, 64, 56, 48, 40, 32, 24, 16, 8):
        if ept % k == 0:
            return k
    raise ValueError(f"edges-per-tile {ept} not divisible by a multiple of 8")


def _sc_mesh():
    return plsc.VectorSubcoreMesh(
        core_axis_name="c", subcore_axis_name="s",
        num_cores=NUM_CORES, num_subcores=NUM_TILES)


def _make_degree(N, E):
    ept = E // NUM_TILES
    K = _pick_chunk(ept)
    nch = ept // K
    slab = (N // NUM_TILES) & ~7
    rem = N - slab * NUM_TILES

    assert nch % 2 == 0

    @functools.partial(
        pl.kernel,
        out_type=jax.ShapeDtypeStruct((NUM_CORES * N, 16), jnp.float32),
        mesh=_sc_mesh(),
        scratch_types=[
            pltpu.VMEM_SHARED((N, 16), jnp.float32),
            pltpu.VMEM((nch, K), jnp.int32),
            pltpu.VMEM((K, 16), jnp.float32),
            pltpu.SemaphoreType.DMA,
        ],
    )
    def deg_kernel(dst_hbm, ones_hbm, zeros_hbm, out_hbm, acc, dst_all,
                   ones_v, sem_a):
        c = lax.axis_index("c")
        s = lax.axis_index("s")
        wid = c * NUM_TILES + s
        pltpu.sync_copy(zeros_hbm.at[pl.ds(0, slab)],
                        acc.at[pl.ds(s * slab, slab)])
        if rem:
            @pl.when(s == 0)
            def _zrem():
                pltpu.sync_copy(zeros_hbm.at[pl.ds(0, rem)],
                                acc.at[pl.ds(NUM_TILES * slab, rem)])
        pltpu.sync_copy(ones_hbm, ones_v)
        pltpu.sync_copy(dst_hbm.at[wid], dst_all)
        plsc.subcore_barrier()

        W = next(w for w in (10, 8, 5, 4, 2, 1) if nch % w == 0)

        def _drain():
            for w in range(W):
                pltpu.make_async_copy(ones_v, acc.at[dst_all.at[0]],
                                      sem_a).wait()

        for w in range(W):
            pltpu.async_copy(ones_v, acc.at[dst_all.at[w]], sem_a, add=True)

        @pl.loop(W, nch, step=W)
        def _window(j):
            for w in range(W):
                pltpu.async_copy(ones_v, acc.at[dst_all.at[j + w]], sem_a,
                                 add=True)
            _drain()

        _drain()

        plsc.subcore_barrier()
        pltpu.sync_copy(acc.at[pl.ds(s * slab, slab)],
                        out_hbm.at[pl.ds(c * N + s * slab, slab)])
        if rem:
            @pl.when(s == 0)
            def _orem():
                pltpu.sync_copy(
                    acc.at[pl.ds(NUM_TILES * slab, rem)],
                    out_hbm.at[pl.ds(c * N + NUM_TILES * slab, rem)])

    return deg_kernel


def _make_segsum(N, E, D):
    ept = E // NUM_TILES
    K = _pick_chunk(ept)
    nch = ept // K
    slab = (N // NUM_TILES) & ~7
    rem = N - slab * NUM_TILES
    assert nch % 3 == 1 and nch >= 7, nch

    @functools.partial(
        pl.kernel,
        out_type=jax.ShapeDtypeStruct((NUM_CORES * N, D), jnp.float32),
        mesh=_sc_mesh(),
        scratch_types=(
            [pltpu.VMEM_SHARED((N, D), jnp.float32)]
            + [pltpu.VMEM((2, K), jnp.int32)] * 3
            + [pltpu.VMEM((K, D), jnp.float32)] * 3
            + [pltpu.SemaphoreType.DMA] * 9
        ),
    )
    def seg_kernel(ys_hbm, comb_hbm, zeros_hbm, out_hbm, acc,
                   idx0, idx1, idx2, buf0, buf1, buf2,
                   si0, si1, si2, sg0, sg1, sg2, ss0, ss1, ss2):
        c = lax.axis_index("c")
        s = lax.axis_index("s")
        pltpu.sync_copy(zeros_hbm.at[pl.ds(0, slab)],
                        acc.at[pl.ds(s * slab, slab)])
        if rem:
            @pl.when(s == 0)
            def _zrem():
                pltpu.sync_copy(zeros_hbm.at[pl.ds(0, rem)],
                                acc.at[pl.ds(NUM_TILES * slab, rem)])
        wid = c * NUM_TILES + s

        idxs = (idx0, idx1, idx2)
        bufs = (buf0, buf1, buf2)
        sis = (si0, si1, si2)
        sgs = (sg0, sg1, sg2)
        sss = (ss0, ss1, ss2)

        def idx_descs(j, l):
            return (pltpu.make_async_copy(comb_hbm.at[wid, j], idxs[l],
                                          sis[l]),)

        def g_desc(l):
            return pltpu.make_async_copy(ys_hbm.at[idxs[l].at[0]], bufs[l],
                                         sgs[l])

        def s_fire(l):
            pltpu.async_copy(bufs[l], acc.at[idxs[l].at[1]], sss[l],
                             add=True)

        def s_wait(l):
            pltpu.make_async_copy(bufs[l], acc.at[idxs[l].at[1]],
                                  sss[l]).wait()

        def chunk(j, l, first=False, prefetch=True):
            m = (l + 2) % 3
            if not first:
                s_wait(m)
            if prefetch:
                for d in idx_descs(j + 2, m):
                    d.start()
            g_desc(l).wait()
            s_fire(l)
            if prefetch:
                for d in idx_descs(j + 2, m):
                    d.wait()
                g_desc(m).start()

        for d in idx_descs(0, 0):
            d.start()
        for d in idx_descs(1, 1):
            d.start()
        plsc.subcore_barrier()
        for d in idx_descs(0, 0):
            d.wait()
        g_desc(0).start()
        for d in idx_descs(1, 1):
            d.wait()
        g_desc(1).start()

        chunk(0, 0, first=True)

        @pl.loop(0, (nch - 7) // 3)
        def _trip(t):
            j = 1 + 3 * t
            chunk(j, 1)
            chunk(j + 1, 2)
            chunk(j + 2, 0)

        for j in range(nch - 6, nch):
            chunk(j, j % 3, prefetch=(j + 2 < nch))
        s_wait((nch - 1) % 3)

        plsc.subcore_barrier()
        pltpu.sync_copy(acc.at[pl.ds(s * slab, slab)],
                        out_hbm.at[pl.ds(c * N + s * slab, slab)])
        if rem:
            @pl.when(s == 0)
            def _orem():
                pltpu.sync_copy(
                    acc.at[pl.ds(NUM_TILES * slab, rem)],
                    out_hbm.at[pl.ds(c * N + NUM_TILES * slab, rem)])

    return seg_kernel


def _mm_prescale_body(x_ref, w_ref, deg_ref, o_ref):
    dinv = lax.rsqrt(deg_ref[:, 0:1] + 1.0)
    o_ref[...] = jnp.dot(x_ref[...], w_ref[...],
                         preferred_element_type=jnp.float32) * dinv


def _mm_prescale(x, w, deg16, br):
    n2, d = x.shape
    grid = (n2 // br,)
    return pl.pallas_call(
        _mm_prescale_body,
        grid=grid,
        in_specs=[
            pl.BlockSpec((br, d), lambda i: (i, 0)),
            pl.BlockSpec((d, w.shape[1]), lambda i: (0, 0)),
            pl.BlockSpec((br, 16), lambda i: (i, 0)),
        ],
        out_specs=pl.BlockSpec((br, w.shape[1]), lambda i: (i, 0)),
        out_shape=jax.ShapeDtypeStruct((n2, w.shape[1]), jnp.float32),
    )(x, w, deg16)


def _post_mm_body(s_ref, ys_ref, b_ref, w_ref, deg_ref, o_ref):
    dinv = lax.rsqrt(deg_ref[:, 0:1] + 1.0)
    h = jnp.maximum(dinv * (s_ref[...] + ys_ref[...]) + b_ref[...], 0.0)
    o_ref[...] = jnp.dot(h, w_ref[...],
                         preferred_element_type=jnp.float32) * dinv


def _post_mm(s1, ys1, b1, w2, deg16, br):
    n2, d = s1.shape
    grid = (n2 // br,)
    return pl.pallas_call(
        _post_mm_body,
        grid=grid,
        in_specs=[
            pl.BlockSpec((br, d), lambda i: (i, 0)),
            pl.BlockSpec((br, d), lambda i: (i, 0)),
            pl.BlockSpec((1, d), lambda i: (0, 0)),
            pl.BlockSpec((d, w2.shape[1]), lambda i: (0, 0)),
            pl.BlockSpec((br, 16), lambda i: (i, 0)),
        ],
        out_specs=pl.BlockSpec((br, w2.shape[1]), lambda i: (i, 0)),
        out_shape=jax.ShapeDtypeStruct((n2, w2.shape[1]), jnp.float32),
    )(s1, ys1, b1, w2, deg16)


def _l2n(x):
    nrm = jnp.sqrt(jnp.sum(x * x, axis=1, keepdims=True))
    return x / jnp.maximum(nrm, 1e-12)


def _flash_body(N, BR, CB, s2_ref, ys2_ref, b2_ref, deg_ref, t_ref,
                out_ref, vgn_ref, tsc_ref, acc_ref):
    i = pl.program_id(0)
    nb = pl.num_programs(0)
    invt = 1.0 / t_ref[0, 0]
    M = jnp.minimum(jnp.abs(invt), 50.0)

    @pl.when(i == 0)
    def _init():
        dinv_v = lax.rsqrt(deg_ref[N:, 0:1] + 1.0)
        ev = dinv_v * (s2_ref[N:, :] + ys2_ref[N:, :]) + b2_ref[...]
        vgn_ref[...] = _l2n(ev).astype(jnp.bfloat16)
        acc_ref[0] = 0.0

    rows = pl.ds(i * BR, BR)
    dinv_t = lax.rsqrt(deg_ref[rows, 0:1] + 1.0)
    et = dinv_t * (s2_ref[rows, :] + ys2_ref[rows, :]) + b2_ref[...]
    tn = _l2n(et)
    tsc_ref[...] = (tn * invt).astype(jnp.bfloat16)
    vg_diag = vgn_ref[rows, :].astype(jnp.float32)
    picked = jnp.clip(jnp.sum(tn * vg_diag, axis=1, keepdims=True) * invt,
                      -50.0, 50.0)

    ones_col = jnp.ones((CB, 1), jnp.bfloat16)

    @pl.loop(0, N // CB, init_carry=jnp.zeros((BR, 1), jnp.float32))
    def col_loop(c, rowsum):
        vc = vgn_ref[pl.ds(c * CB, CB), :]
        simc = lax.dot_general(tsc_ref[...], vc, (((1,), (1,)), ((), ())),
                               preferred_element_type=jnp.float32)
        z = (jnp.clip(simc, -50.0, 50.0) - M).astype(jnp.bfloat16)
        e = jnp.exp(z)
        return rowsum + jnp.dot(e, ones_col,
                                preferred_element_type=jnp.float32)

    logz = M + jnp.log(col_loop)
    acc_ref[0] += jnp.sum(logz - picked)

    @pl.when(i == nb - 1)
    def _fin():
        out_ref[0, 0] = acc_ref[0] * (1.0 / N)


def _flash_loss(s2, ys2, b2, deg16, temp, N, BR, CB):
    d = s2.shape[1]
    grid = (N // BR,)
    return pl.pallas_call(
        functools.partial(_flash_body, N, BR, CB),
        grid=grid,
        in_specs=[
            pl.BlockSpec((2 * N, d), lambda i: (0, 0)),
            pl.BlockSpec((2 * N, d), lambda i: (0, 0)),
            pl.BlockSpec((1, d), lambda i: (0, 0)),
            pl.BlockSpec((2 * N, 16), lambda i: (0, 0)),
            pl.BlockSpec(memory_space=pltpu.SMEM),
        ],
        out_specs=pl.BlockSpec((1, 1), lambda i: (0, 0),
                               memory_space=pltpu.SMEM),
        out_shape=jax.ShapeDtypeStruct((1, 1), jnp.float32),
        scratch_shapes=[
            pltpu.VMEM((N, d), jnp.bfloat16),
            pltpu.VMEM((BR, d), jnp.bfloat16),
            pltpu.SMEM((1,), jnp.float32),
        ],
    )(s2, ys2, b2, deg16, temp)


def kernel(tg_x, tg_edge_index, vg_x, vg_edge_index, W1, b1, W2, b2,
           temperature):
    N, D = tg_x.shape
    E = tg_edge_index.shape[1]

    X = jnp.concatenate([tg_x, vg_x], axis=0)
    src_g = jnp.concatenate([tg_edge_index[0], vg_edge_index[0] + N])
    dst_l = jnp.concatenate([tg_edge_index[1], vg_edge_index[1]])

    npt = N // NUM_TILES
    ept = E // NUM_TILES
    Kd = _pick_chunk(ept)
    dst3 = dst_l.reshape(NUM_CORES * NUM_TILES, ept // Kd, Kd)
    zeros16 = jnp.zeros((npt, 16), jnp.float32)
    ones16 = jnp.ones((Kd, 16), jnp.float32)
    zerosD = jnp.zeros((npt, D), jnp.float32)
    b1r = b1.reshape(1, -1)
    b2r = b2.reshape(1, -1)
    tempr = jnp.asarray(temperature, jnp.float32).reshape(1, 1)

    nchs = ept // Kd
    comb = jnp.concatenate(
        [src_g.reshape(NUM_CORES * NUM_TILES, nchs, 1, Kd),
         dst_l.reshape(NUM_CORES * NUM_TILES, nchs, 1, Kd)], axis=2)

    deg16 = _make_degree(N, E)(dst3, ones16, zeros16)
    segsum = _make_segsum(N, E, D)

    ys1 = _mm_prescale(X, W1, deg16, br=1000)
    s1 = segsum(ys1, comb, zerosD)
    ys2 = _post_mm(s1, ys1, b1r, W2, deg16, br=1000)
    s2 = segsum(ys2, comb, zerosD)
    loss = _flash_loss(s2, ys2, b2r, deg16, tempr, N, BR=1000, CB=2000)
    return loss[0, 0]


# speedup vs baseline: 1.0367x; 1.0081x over previous
import functools

import jax
import jax.numpy as jnp
from jax import lax
from jax.experimental import pallas as pl
from jax.experimental.pallas import tpu as pltpu
from jax.experimental.pallas import tpu_sc as plsc

NUM_CORES = 2
NUM_TILES = 16


def _pick_chunk(ept):
    for k in (128, 120, 112, 104, 96, 88, 80, 72, 64, 56, 48, 40, 32, 24, 16, 8):
        if ept % k == 0:
            return k
    raise ValueError(f"edges-per-tile {ept} not divisible by a multiple of 8")


def _sc_mesh():
    return plsc.VectorSubcoreMesh(
        core_axis_name="c", subcore_axis_name="s",
        num_cores=NUM_CORES, num_subcores=NUM_TILES)


def _make_degree(N, E):
    ept = E // NUM_TILES
    K = _pick_chunk(ept)
    nch = ept // K
    slab = (N // NUM_TILES) & ~7
    rem = N - slab * NUM_TILES

    assert nch % 2 == 0

    @functools.partial(
        pl.kernel,
        out_type=jax.ShapeDtypeStruct((NUM_CORES * N, 16), jnp.float32),
        mesh=_sc_mesh(),
        scratch_types=[
            pltpu.VMEM_SHARED((N, 16), jnp.float32),
            pltpu.VMEM((nch, K), jnp.int32),
            pltpu.VMEM((K, 16), jnp.float32),
            pltpu.SemaphoreType.DMA,
        ],
    )
    def deg_kernel(dst_hbm, ones_hbm, zeros_hbm, out_hbm, acc, dst_all,
                   ones_v, sem_a):
        c = lax.axis_index("c")
        s = lax.axis_index("s")
        wid = c * NUM_TILES + s
        pltpu.sync_copy(zeros_hbm.at[pl.ds(0, slab)],
                        acc.at[pl.ds(s * slab, slab)])
        if rem:
            @pl.when(s == 0)
            def _zrem():
                pltpu.sync_copy(zeros_hbm.at[pl.ds(0, rem)],
                                acc.at[pl.ds(NUM_TILES * slab, rem)])
        pltpu.sync_copy(ones_hbm, ones_v)
        pltpu.sync_copy(dst_hbm.at[wid], dst_all)
        plsc.subcore_barrier()

        W = next(w for w in (10, 8, 5, 4, 2, 1) if nch % w == 0)

        def _drain():
            for w in range(W):
                pltpu.make_async_copy(ones_v, acc.at[dst_all.at[0]],
                                      sem_a).wait()

        for w in range(W):
            pltpu.async_copy(ones_v, acc.at[dst_all.at[w]], sem_a, add=True)

        @pl.loop(W, nch, step=W)
        def _window(j):
            for w in range(W):
                pltpu.async_copy(ones_v, acc.at[dst_all.at[j + w]], sem_a,
                                 add=True)
            _drain()

        _drain()

        plsc.subcore_barrier()
        pltpu.sync_copy(acc.at[pl.ds(s * slab, slab)],
                        out_hbm.at[pl.ds(c * N + s * slab, slab)])
        if rem:
            @pl.when(s == 0)
            def _orem():
                pltpu.sync_copy(
                    acc.at[pl.ds(NUM_TILES * slab, rem)],
                    out_hbm.at[pl.ds(c * N + NUM_TILES * slab, rem)])

    return deg_kernel


def _make_segsum(N, E, D):
    ept = E // NUM_TILES
    K = _pick_chunk(ept)
    nch = ept // K
    slab = (N // NUM_TILES) & ~7
    rem = N - slab * NUM_TILES
    assert nch % 3 == 1 and nch >= 7, nch

    @functools.partial(
        pl.kernel,
        out_type=jax.ShapeDtypeStruct((NUM_CORES * N, D), jnp.float32),
        mesh=_sc_mesh(),
        scratch_types=(
            [pltpu.VMEM_SHARED((N, D), jnp.float32)]
            + [pltpu.VMEM((2, K), jnp.int32)] * 3
            + [pltpu.VMEM((K, D), jnp.float32)] * 3
            + [pltpu.SemaphoreType.DMA] * 9
        ),
    )
    def seg_kernel(ys_hbm, comb_hbm, zeros_hbm, out_hbm, acc,
                   idx0, idx1, idx2, buf0, buf1, buf2,
                   si0, si1, si2, sg0, sg1, sg2, ss0, ss1, ss2):
        c = lax.axis_index("c")
        s = lax.axis_index("s")
        pltpu.sync_copy(zeros_hbm.at[pl.ds(0, slab)],
                        acc.at[pl.ds(s * slab, slab)])
        if rem:
            @pl.when(s == 0)
            def _zrem():
                pltpu.sync_copy(zeros_hbm.at[pl.ds(0, rem)],
                                acc.at[pl.ds(NUM_TILES * slab, rem)])
        wid = c * NUM_TILES + s

        idxs = (idx0, idx1, idx2)
        bufs = (buf0, buf1, buf2)
        sis = (si0, si1, si2)
        sgs = (sg0, sg1, sg2)
        sss = (ss0, ss1, ss2)

        def idx_descs(j, l):
            return (pltpu.make_async_copy(comb_hbm.at[wid, j], idxs[l],
                                          sis[l]),)

        def g_desc(l):
            return pltpu.make_async_copy(ys_hbm.at[idxs[l].at[0]], bufs[l],
                                         sgs[l])

        def s_fire(l):
            pltpu.async_copy(bufs[l], acc.at[idxs[l].at[1]], sss[l],
                             add=True)

        def s_wait(l):
            pltpu.make_async_copy(bufs[l], acc.at[idxs[l].at[1]],
                                  sss[l]).wait()

        def chunk(j, l, first=False, prefetch=True):
            m = (l + 2) % 3
            if not first:
                s_wait(m)
            if prefetch:
                for d in idx_descs(j + 2, m):
                    d.start()
            g_desc(l).wait()
            s_fire(l)
            if prefetch:
                for d in idx_descs(j + 2, m):
                    d.wait()
                g_desc(m).start()

        for d in idx_descs(0, 0):
            d.start()
        for d in idx_descs(1, 1):
            d.start()
        plsc.subcore_barrier()
        for d in idx_descs(0, 0):
            d.wait()
        g_desc(0).start()
        for d in idx_descs(1, 1):
            d.wait()
        g_desc(1).start()

        chunk(0, 0, first=True)

        @pl.loop(0, (nch - 7) // 3)
        def _trip(t):
            j = 1 + 3 * t
            chunk(j, 1)
            chunk(j + 1, 2)
            chunk(j + 2, 0)

        for j in range(nch - 6, nch):
            chunk(j, j % 3, prefetch=(j + 2 < nch))
        s_wait((nch - 1) % 3)

        plsc.subcore_barrier()
        pltpu.sync_copy(acc.at[pl.ds(s * slab, slab)],
                        out_hbm.at[pl.ds(c * N + s * slab, slab)])
        if rem:
            @pl.when(s == 0)
            def _orem():
                pltpu.sync_copy(
                    acc.at[pl.ds(NUM_TILES * slab, rem)],
                    out_hbm.at[pl.ds(c * N + NUM_TILES * slab, rem)])

    return seg_kernel


def _mm_prescale_body(x_ref, w_ref, deg_ref, o_ref):
    dinv = lax.rsqrt(deg_ref[:, 0:1] + 1.0)
    o_ref[...] = jnp.dot(x_ref[...], w_ref[...],
                         preferred_element_type=jnp.float32) * dinv


def _mm_prescale(x, w, deg16, br):
    n2, d = x.shape
    grid = (n2 // br,)
    return pl.pallas_call(
        _mm_prescale_body,
        grid=grid,
        in_specs=[
            pl.BlockSpec((br, d), lambda i: (i, 0)),
            pl.BlockSpec((d, w.shape[1]), lambda i: (0, 0)),
            pl.BlockSpec((br, 16), lambda i: (i, 0)),
        ],
        out_specs=pl.BlockSpec((br, w.shape[1]), lambda i: (i, 0)),
        out_shape=jax.ShapeDtypeStruct((n2, w.shape[1]), jnp.float32),
    )(x, w, deg16)


def _post_mm_body(s_ref, ys_ref, b_ref, w_ref, deg_ref, o_ref):
    dinv = lax.rsqrt(deg_ref[:, 0:1] + 1.0)
    h = jnp.maximum(dinv * (s_ref[...] + ys_ref[...]) + b_ref[...], 0.0)
    o_ref[...] = jnp.dot(h, w_ref[...],
                         preferred_element_type=jnp.float32) * dinv


def _post_mm(s1, ys1, b1, w2, deg16, br):
    n2, d = s1.shape
    grid = (n2 // br,)
    return pl.pallas_call(
        _post_mm_body,
        grid=grid,
        in_specs=[
            pl.BlockSpec((br, d), lambda i: (i, 0)),
            pl.BlockSpec((br, d), lambda i: (i, 0)),
            pl.BlockSpec((1, d), lambda i: (0, 0)),
            pl.BlockSpec((d, w2.shape[1]), lambda i: (0, 0)),
            pl.BlockSpec((br, 16), lambda i: (i, 0)),
        ],
        out_specs=pl.BlockSpec((br, w2.shape[1]), lambda i: (i, 0)),
        out_shape=jax.ShapeDtypeStruct((n2, w2.shape[1]), jnp.float32),
    )(s1, ys1, b1, w2, deg16)


def _l2n(x):
    nrm = jnp.sqrt(jnp.sum(x * x, axis=1, keepdims=True))
    return x / jnp.maximum(nrm, 1e-12)


def _flash_body(N, BR, CB, s2_ref, ys2_ref, b2_ref, deg_ref, t_ref,
                out_ref, vgn_ref, tsc_ref, acc_ref):
    i = pl.program_id(0)
    nb = pl.num_programs(0)
    invt = 1.0 / t_ref[0, 0]
    M = jnp.minimum(jnp.abs(invt), 50.0)

    @pl.when(i == 0)
    def _init():
        dinv_v = lax.rsqrt(deg_ref[N:, 0:1] + 1.0)
        ev = dinv_v * (s2_ref[N:, :] + ys2_ref[N:, :]) + b2_ref[...]
        vgn_ref[...] = _l2n(ev).astype(jnp.bfloat16)
        acc_ref[0] = 0.0

    rows = pl.ds(i * BR, BR)
    dinv_t = lax.rsqrt(deg_ref[rows, 0:1] + 1.0)
    et = dinv_t * (s2_ref[rows, :] + ys2_ref[rows, :]) + b2_ref[...]
    tn = _l2n(et)
    tsc_ref[...] = (tn * invt).astype(jnp.bfloat16)
    vg_diag = vgn_ref[rows, :].astype(jnp.float32)
    picked = jnp.clip(jnp.sum(tn * vg_diag, axis=1, keepdims=True) * invt,
                      -50.0, 50.0)

    ones_col = jnp.ones((CB, 1), jnp.bfloat16)

    @pl.loop(0, N // CB, init_carry=jnp.zeros((BR, 1), jnp.float32))
    def col_loop(c, rowsum):
        vc = vgn_ref[pl.ds(c * CB, CB), :]
        simc = lax.dot_general(tsc_ref[...], vc, (((1,), (1,)), ((), ())),
                               preferred_element_type=jnp.float32)
        z = (jnp.clip(simc, -50.0, 50.0) - M).astype(jnp.bfloat16)
        e = jnp.exp(z)
        return rowsum + jnp.dot(e, ones_col,
                                preferred_element_type=jnp.float32)

    logz = M + jnp.log(col_loop)
    acc_ref[0] += jnp.sum(logz - picked)

    @pl.when(i == nb - 1)
    def _fin():
        out_ref[0, 0] = acc_ref[0] * (1.0 / N)


def _flash_loss(s2, ys2, b2, deg16, temp, N, BR, CB):
    d = s2.shape[1]
    grid = (N // BR,)
    return pl.pallas_call(
        functools.partial(_flash_body, N, BR, CB),
        grid=grid,
        in_specs=[
            pl.BlockSpec((2 * N, d), lambda i: (0, 0)),
            pl.BlockSpec((2 * N, d), lambda i: (0, 0)),
            pl.BlockSpec((1, d), lambda i: (0, 0)),
            pl.BlockSpec((2 * N, 16), lambda i: (0, 0)),
            pl.BlockSpec(memory_space=pltpu.SMEM),
        ],
        out_specs=pl.BlockSpec((1, 1), lambda i: (0, 0),
                               memory_space=pltpu.SMEM),
        out_shape=jax.ShapeDtypeStruct((1, 1), jnp.float32),
        scratch_shapes=[
            pltpu.VMEM((N, d), jnp.bfloat16),
            pltpu.VMEM((BR, d), jnp.bfloat16),
            pltpu.SMEM((1,), jnp.float32),
        ],
    )(s2, ys2, b2, deg16, temp)


def kernel(tg_x, tg_edge_index, vg_x, vg_edge_index, W1, b1, W2, b2,
           temperature):
    N, D = tg_x.shape
    E = tg_edge_index.shape[1]

    X = jnp.concatenate([tg_x, vg_x], axis=0)
    src_g = jnp.concatenate([tg_edge_index[0], vg_edge_index[0] + N])
    dst_l = jnp.concatenate([tg_edge_index[1], vg_edge_index[1]])

    npt = N // NUM_TILES
    ept = E // NUM_TILES
    Kd = _pick_chunk(ept)
    dst3 = dst_l.reshape(NUM_CORES * NUM_TILES, ept // Kd, Kd)
    zeros16 = jnp.zeros((npt, 16), jnp.float32)
    ones16 = jnp.ones((Kd, 16), jnp.float32)
    zerosD = jnp.zeros((npt, D), jnp.float32)
    b1r = b1.reshape(1, -1)
    b2r = b2.reshape(1, -1)
    tempr = jnp.asarray(temperature, jnp.float32).reshape(1, 1)

    nchs = ept // Kd
    comb = jnp.concatenate(
        [src_g.reshape(NUM_CORES * NUM_TILES, nchs, 1, Kd),
         dst_l.reshape(NUM_CORES * NUM_TILES, nchs, 1, Kd)], axis=2)

    deg16 = _make_degree(N, E)(dst3, ones16, zeros16)
    segsum = _make_segsum(N, E, D)

    ys1 = _mm_prescale(X, W1, deg16, br=1000)
    s1 = segsum(ys1, comb, zerosD)
    ys2 = _post_mm(s1, ys1, b1r, W2, deg16, br=1000)
    s2 = segsum(ys2, comb, zerosD)
    loss = _flash_loss(s2, ys2, b2r, deg16, tempr, N, BR=2000, CB=2000)
    return loss[0, 0]
